# baseline (speedup 1.0000x reference)
import jax
import jax.numpy as jnp
from jax import lax
from jax.experimental import pallas as pl
from jax.experimental.pallas import tpu as pltpu
from jax.experimental.pallas import tpu_sc as plsc

_BATCH = 4
_NT = 8192
_TS = 768
_NC = 2
_NS = 16
_NW = _NC * _NS
_ROWS_W = _NT // _NW
_CHUNK = 32
_NCHUNK = _ROWS_W // _CHUNK
_VPR = _TS // 16
_NSTEP = _NCHUNK * _BATCH


def _sc_body(x_hbm, t_hbm, o_hbm,
             xbuf0, xbuf1, tbuf0, tbuf1,
             xin0, xin1, xout0, xout1, tin0, tin1):
    wid = lax.axis_index("s") * _NC + lax.axis_index("c")
    row0 = wid * _ROWS_W

    xbufs = [xbuf0, xbuf1]
    tbufs = [tbuf0, tbuf1]
    xin = [xin0, xin1]
    xout = [xout0, xout1]
    tin = [tin0, tin1]

    def start_xload(s):
        c, b = divmod(s, _BATCH)
        r = row0 + c * _CHUNK
        return pltpu.async_copy(
            x_hbm.at[b, pl.ds(r, _CHUNK), :], xbufs[s % 2], xin[s % 2])

    def start_tload(c):
        r = row0 + c * _CHUNK
        return pltpu.async_copy(
            t_hbm.at[pl.ds(r, _CHUNK), :], tbufs[c % 2], tin[c % 2])

    tloads = {0: start_tload(0)}
    xloads = {0: start_xload(0)}
    stores = {}

    for s in range(_NSTEP):
        c, b = divmod(s, _BATCH)
        buf = s % 2
        if b == 0:
            tloads.pop(c).wait()
            if c + 1 < _NCHUNK:
                tloads[c + 1] = start_tload(c + 1)
        if s + 1 < _NSTEP:
            if s - 1 in stores:
                stores.pop(s - 1).wait()
            xloads[s + 1] = start_xload(s + 1)
        xloads.pop(s).wait()

        xb = xbufs[buf]
        tb = tbufs[c % 2]

        def _row_body(r, carry):
            @plsc.parallel_loop(0, _TS, step=16, unroll=8)
            def _(u):
                sl = pl.ds(u, 16)
                plsc.addupdate(xb.at[r, sl], tb[r, sl])
            return carry

        lax.fori_loop(0, _CHUNK, _row_body, 0)

        cc, bb = divmod(s, _BATCH)
        r = row0 + cc * _CHUNK
        stores[s] = pltpu.async_copy(
            xb, o_hbm.at[bb, pl.ds(r, _CHUNK), :], xout[buf])

    for s in list(stores):
        stores.pop(s).wait()


def kernel(x, table):
    mesh = plsc.VectorSubcoreMesh(core_axis_name="c", subcore_axis_name="s")
    k = pl.kernel(
        _sc_body,
        out_type=jax.ShapeDtypeStruct((_BATCH, _NT, _TS), jnp.float32),
        mesh=mesh,
        scratch_types=[
            pltpu.VMEM((_CHUNK, _TS), jnp.float32),
            pltpu.VMEM((_CHUNK, _TS), jnp.float32),
            pltpu.VMEM((_CHUNK, _TS), jnp.float32),
            pltpu.VMEM((_CHUNK, _TS), jnp.float32),
            pltpu.SemaphoreType.DMA,
            pltpu.SemaphoreType.DMA,
            pltpu.SemaphoreType.DMA,
            pltpu.SemaphoreType.DMA,
            pltpu.SemaphoreType.DMA,
            pltpu.SemaphoreType.DMA,
        ],
    )
    return k(x, table)


# speedup vs baseline: 1.1353x; 1.1353x over previous
import jax
import jax.numpy as jnp
from jax import lax
from jax.experimental import pallas as pl
from jax.experimental.pallas import tpu as pltpu
from jax.experimental.pallas import tpu_sc as plsc

_BATCH = 4
_NT = 8192
_TS = 768
_NC = 2
_NS = 16
_NW = _NC * _NS
_ROWS_W = _NT // _NW
_CHUNK = 32
_NCHUNK = _ROWS_W // _CHUNK
_NSTEP = _NCHUNK * _BATCH
_NBUF = 3


def _sc_body(x_hbm, t_hbm, o_hbm,
             xbuf0, xbuf1, xbuf2, tbuf0, tbuf1,
             xin0, xin1, xin2, xout0, xout1, xout2, tin0, tin1):
    wid = lax.axis_index("s") * _NC + lax.axis_index("c")
    row0 = wid * _ROWS_W

    xbufs = [xbuf0, xbuf1, xbuf2]
    tbufs = [tbuf0, tbuf1]
    xin = [xin0, xin1, xin2]
    xout = [xout0, xout1, xout2]
    tin = [tin0, tin1]

    def start_xload(s):
        c, b = divmod(s, _BATCH)
        r = row0 + c * _CHUNK
        i = s % _NBUF
        return pltpu.async_copy(
            x_hbm.at[b, pl.ds(r, _CHUNK), :], xbufs[i], xin[i])

    def start_tload(c):
        r = row0 + c * _CHUNK
        return pltpu.async_copy(
            t_hbm.at[pl.ds(r, _CHUNK), :], tbufs[c % 2], tin[c % 2])

    tloads = {0: start_tload(0)}
    xloads = {0: start_xload(0)}
    stores = {}

    for s in range(_NSTEP):
        c, b = divmod(s, _BATCH)
        i = s % _NBUF
        if b == 0:
            tloads.pop(c).wait()
            if c + 1 < _NCHUNK:
                tloads[c + 1] = start_tload(c + 1)
        if s + 1 < _NSTEP:
            if s - 2 in stores:
                stores.pop(s - 2).wait()
            xloads[s + 1] = start_xload(s + 1)
        xloads.pop(s).wait()

        xb = xbufs[i]
        tb = tbufs[c % 2]

        @plsc.parallel_loop(0, _CHUNK, step=1)
        def _(r):
            @plsc.parallel_loop(0, _TS, step=16, unroll=8)
            def _(u):
                sl = pl.ds(u, 16)
                plsc.addupdate(xb.at[r, sl], tb[r, sl])

        r = row0 + c * _CHUNK
        stores[s] = pltpu.async_copy(
            xb, o_hbm.at[b, pl.ds(r, _CHUNK), :], xout[i])

    for s in list(stores):
        stores.pop(s).wait()


def kernel(x, table):
    mesh = plsc.VectorSubcoreMesh(core_axis_name="c", subcore_axis_name="s")
    k = pl.kernel(
        _sc_body,
        out_type=jax.ShapeDtypeStruct((_BATCH, _NT, _TS), jnp.float32),
        mesh=mesh,
        scratch_types=[
            pltpu.VMEM((_CHUNK, _TS), jnp.float32),
            pltpu.VMEM((_CHUNK, _TS), jnp.float32),
            pltpu.VMEM((_CHUNK, _TS), jnp.float32),
            pltpu.VMEM((_CHUNK, _TS), jnp.float32),
            pltpu.VMEM((_CHUNK, _TS), jnp.float32),
            pltpu.SemaphoreType.DMA,
            pltpu.SemaphoreType.DMA,
            pltpu.SemaphoreType.DMA,
            pltpu.SemaphoreType.DMA,
            pltpu.SemaphoreType.DMA,
            pltpu.SemaphoreType.DMA,
            pltpu.SemaphoreType.DMA,
            pltpu.SemaphoreType.DMA,
        ],
    )
    return k(x, table)
